# baseline (speedup 1.0000x reference)
import functools

import jax
import jax.numpy as jnp
from jax import lax
from jax.experimental import pallas as pl
from jax.experimental.pallas import tpu as pltpu
from jax.experimental.pallas import tpu_sc as plsc

ND = 50000
NP_ = 50176
CHUNK = 12544
CT = CHUNK + 8
TPB = CHUNK // 16
EB = 128
BM = 256
GPAD = 100352
GB = GPAD // BM


@functools.lru_cache(maxsize=None)
def _agg_call(epad):
    epw = epad // 16
    nsteps = epw // EB
    mesh = plsc.VectorSubcoreMesh(core_axis_name="c", subcore_axis_name="s")

    @functools.partial(
        pl.kernel,
        mesh=mesh,
        out_type=jax.ShapeDtypeStruct((NP_, 128), jnp.float32),
        scratch_types=[
            pltpu.VMEM((2 * EB,), jnp.int32),
            pltpu.VMEM((EB,), jnp.int32),
            pltpu.VMEM((EB, 128), jnp.float32),
            pltpu.VMEM((48, 128), jnp.float32),
            pltpu.VMEM_SHARED((CT, 128), jnp.float32),
            pltpu.SemaphoreType.DMA,
        ],
    )
    def agg(sd_h, y_h, out_h, sd_v, dl_v, rows_v, zb_v, agg_s, sem):
        core = lax.axis_index("c")
        sub = lax.axis_index("s")
        ebase = sub * epw * 2
        zero16 = jnp.zeros((16,), jnp.float32)

        def zrow(i, c):
            for j in range(8):
                zb_v[i, pl.ds(j * 16, 16)] = zero16
            return c

        lax.fori_loop(0, 48, zrow, 0)
        for p in range(2):
            lo = (p * 2 + core) * CHUNK
            for t in range(16):
                pltpu.sync_copy(zb_v, agg_s.at[pl.ds(sub * TPB + t * 48, 48)])
            pltpu.sync_copy(zb_v.at[pl.ds(0, 16)],
                            agg_s.at[pl.ds(sub * TPB + 768, 16)])
            plsc.subcore_barrier()

            def step(i, carry):
                off = ebase + i * (2 * EB)
                pltpu.sync_copy(sd_h.at[pl.ds(off, 2 * EB)], sd_v)
                pltpu.async_copy(y_h.at[sd_v.at[pl.ds(0, EB)]], rows_v,
                                 sem).wait()
                for j in range(8):
                    d16 = sd_v[pl.ds(EB + j * 16, 16)]
                    m = (d16 >= lo) & (d16 < lo + CHUNK)
                    dl_v[pl.ds(j * 16, 16)] = jnp.where(m, d16 - lo, CHUNK)
                pltpu.sync_copy(rows_v, agg_s.at[dl_v], add=True)
                return carry

            lax.fori_loop(0, nsteps, step, 0)
            plsc.subcore_barrier()
            pltpu.sync_copy(
                agg_s.at[pl.ds(sub * TPB, TPB)],
                out_h.at[pl.ds(lo + sub * TPB, TPB)],
            )
            plsc.subcore_barrier()

    return agg


@functools.lru_cache(maxsize=None)
def _gather_call(n):
    per_w = n // 32
    nsteps = per_w // EB
    mesh = plsc.VectorSubcoreMesh(core_axis_name="c", subcore_axis_name="s")

    npairs = nsteps // 2
    assert nsteps % 2 == 0

    @functools.partial(
        pl.kernel,
        mesh=mesh,
        out_type=jax.ShapeDtypeStruct((n, 128), jnp.float32),
        scratch_types=[
            pltpu.VMEM((EB,), jnp.int32), pltpu.VMEM((EB,), jnp.int32),
            pltpu.VMEM((EB, 128), jnp.float32),
            pltpu.VMEM((EB, 128), jnp.float32),
            pltpu.SemaphoreType.DMA, pltpu.SemaphoreType.DMA,
            pltpu.SemaphoreType.DMA, pltpu.SemaphoreType.DMA,
            pltpu.SemaphoreType.DMA, pltpu.SemaphoreType.DMA,
        ],
    )
    def gat(idx_h, z_h, out_h, i0, i1, rows0, rows1,
            si0, si1, sg0, sg1, so0, so1):
        core = lax.axis_index("c")
        sub = lax.axis_index("s")
        base = (sub * 2 + core) * per_w
        pltpu.async_copy(idx_h.at[pl.ds(base, EB)], i0, si0)
        pltpu.make_async_copy(idx_h.at[pl.ds(base, EB)], i0, si0).wait()
        pltpu.async_copy(z_h.at[i0], rows0, sg0)

        def pair(k, first):
            off0 = base + 2 * k * EB
            off1 = off0 + EB
            off2 = off0 + 2 * EB
            pltpu.async_copy(idx_h.at[pl.ds(off1, EB)], i1, si1)
            pltpu.make_async_copy(z_h.at[i0], rows0, sg0).wait()
            pltpu.async_copy(rows0, out_h.at[pl.ds(off0, EB)], so0)
            pltpu.make_async_copy(idx_h.at[pl.ds(off1, EB)], i1, si1).wait()
            if not first:
                pltpu.make_async_copy(rows1, out_h.at[pl.ds(off1, EB)],
                                      so1).wait()
            pltpu.async_copy(z_h.at[i1], rows1, sg1)
            pltpu.async_copy(idx_h.at[pl.ds(off2, EB)], i0, si0)
            pltpu.make_async_copy(z_h.at[i1], rows1, sg1).wait()
            pltpu.async_copy(rows1, out_h.at[pl.ds(off1, EB)], so1)
            pltpu.make_async_copy(idx_h.at[pl.ds(off2, EB)], i0, si0).wait()
            pltpu.make_async_copy(rows0, out_h.at[pl.ds(off0, EB)], so0).wait()
            pltpu.async_copy(z_h.at[i0], rows0, sg0)

        pair(0, True)
        lax.fori_loop(1, npairs, lambda k, c: (pair(k, False), c)[1], 0)
        pltpu.make_async_copy(z_h.at[i0], rows0, sg0).wait()
        pltpu.make_async_copy(rows1, out_h.at[pl.ds(base, EB)], so1).wait()

    return gat


@functools.lru_cache(maxsize=None)
def _mm_call(k):
    def body(x_ref, w_ref, b_ref, *o_refs):
        x = x_ref[...]
        for t in range(k):
            o_refs[t][...] = (
                jnp.dot(x, w_ref[:, t * 128:(t + 1) * 128],
                        preferred_element_type=jnp.float32)
                + b_ref[0, t * 128:(t + 1) * 128][None, :]
            )

    return pl.pallas_call(
        body,
        grid=(NP_ // BM,),
        in_specs=[
            pl.BlockSpec((BM, 128), lambda i: (i, 0)),
            pl.BlockSpec((128, 128 * k), lambda i: (0, 0)),
            pl.BlockSpec((8, 128 * k), lambda i: (0, 0)),
        ],
        out_specs=[pl.BlockSpec((BM, 128), lambda i: (i, 0))] * k,
        out_shape=[jax.ShapeDtypeStruct((NP_, 128), jnp.float32)] * k,
    )


@functools.lru_cache(maxsize=None)
def _comb_call(nt, relu):
    def body(*refs):
        o_ref = refs[-1]
        acc = None
        for t in range(nt):
            a = refs[2 * t][...] + refs[2 * t + 1][...]
            n2 = jnp.sum(a * a, axis=1, keepdims=True)
            a = a / jnp.maximum(jnp.sqrt(n2), 1e-12)
            acc = a if acc is None else acc + a
        if relu:
            acc = jnp.maximum(acc, 0.0)
        o_ref[...] = acc

    return pl.pallas_call(
        body,
        grid=(NP_ // BM,),
        in_specs=[pl.BlockSpec((BM, 128), lambda i: (i, 0))] * (2 * nt),
        out_specs=pl.BlockSpec((BM, 128), lambda i: (i, 0)),
        out_shape=jax.ShapeDtypeStruct((NP_, 128), jnp.float32),
    )


@functools.lru_cache(maxsize=None)
def _dec_call(blk_a, blk_b):
    DB = 8 * BM

    def body(za_ref, zb_ref, r_ref, o_ref):
        t = jnp.dot(za_ref[...], r_ref[...],
                    preferred_element_type=jnp.float32) * zb_ref[...]
        o_ref[...] = jnp.sum(t, axis=1).reshape(8, BM)

    return pl.pallas_call(
        body,
        grid=(GPAD // DB,),
        in_specs=[
            pl.BlockSpec((DB, 128), lambda i, o=blk_a: (i + o, 0)),
            pl.BlockSpec((DB, 128), lambda i, o=blk_b: (i + o, 0)),
            pl.BlockSpec((128, 128), lambda i: (0, 0)),
        ],
        out_specs=pl.BlockSpec((8, BM), lambda i: (i, 0)),
        out_shape=jax.ShapeDtypeStruct((GB, BM), jnp.float32),
    )


def _pad_rows(x):
    return jnp.concatenate(
        [x, jnp.zeros((NP_ - x.shape[0], x.shape[1]), x.dtype)], axis=0)


def _pad_edges(ei, epad):
    e = ei.shape[1]
    src = jnp.concatenate(
        [ei[0].astype(jnp.int32), jnp.zeros((epad - e,), jnp.int32)])
    dst = jnp.concatenate(
        [ei[1].astype(jnp.int32), jnp.full((epad - e,), ND, jnp.int32)])
    return jnp.stack(
        [src.reshape(-1, EB), dst.reshape(-1, EB)], axis=1).reshape(-1)


def _pad_idx(ix, extra=0):
    return jnp.concatenate(
        [ix.astype(jnp.int32),
         jnp.zeros((GPAD + extra - ix.shape[0],), jnp.int32)])


def kernel(x_drug, x_gene, ei_gene_interact_gene, ei_drug_has_target_gene, ei_gene_get_target_drug, ei_drug_rel0_drug, ei_drug_rel1_drug, Wm_gene_interact_gene_0, bm_gene_interact_gene_0, Ws_gene_interact_gene_0, bs_gene_interact_gene_0, Wm_drug_has_target_gene_0, bm_drug_has_target_gene_0, Ws_drug_has_target_gene_0, bs_drug_has_target_gene_0, Wm_gene_get_target_drug_0, bm_gene_get_target_drug_0, Ws_gene_get_target_drug_0, bs_gene_get_target_drug_0, Wm_drug_rel0_drug_0, bm_drug_rel0_drug_0, Ws_drug_rel0_drug_0, bs_drug_rel0_drug_0, Wm_drug_rel1_drug_0, bm_drug_rel1_drug_0, Ws_drug_rel1_drug_0, bs_drug_rel1_drug_0, Wm_gene_interact_gene_1, bm_gene_interact_gene_1, Ws_gene_interact_gene_1, bs_gene_interact_gene_1, Wm_drug_has_target_gene_1, bm_drug_has_target_gene_1, Ws_drug_has_target_gene_1, bs_drug_has_target_gene_1, Wm_gene_get_target_drug_1, bm_gene_get_target_drug_1, Ws_gene_get_target_drug_1, bs_gene_get_target_drug_1, Wm_drug_rel0_drug_1, bm_drug_rel0_drug_1, Ws_drug_rel0_drug_1, bs_drug_rel0_drug_1, Wm_drug_rel1_drug_1, bm_drug_rel1_drug_1, Ws_drug_rel1_drug_1, bs_drug_rel1_drug_1, R_bilinear_rel0, R_dedicom, D_dedicom_rel1):
    Wl = [
        (Wm_gene_interact_gene_0, bm_gene_interact_gene_0,
         Ws_gene_interact_gene_0, bs_gene_interact_gene_0,
         Wm_drug_has_target_gene_0, bm_drug_has_target_gene_0,
         Ws_drug_has_target_gene_0, bs_drug_has_target_gene_0,
         Wm_gene_get_target_drug_0, bm_gene_get_target_drug_0,
         Ws_gene_get_target_drug_0, bs_gene_get_target_drug_0,
         Wm_drug_rel0_drug_0, bm_drug_rel0_drug_0,
         Ws_drug_rel0_drug_0, bs_drug_rel0_drug_0,
         Wm_drug_rel1_drug_0, bm_drug_rel1_drug_0,
         Ws_drug_rel1_drug_0, bs_drug_rel1_drug_0),
        (Wm_gene_interact_gene_1, bm_gene_interact_gene_1,
         Ws_gene_interact_gene_1, bs_gene_interact_gene_1,
         Wm_drug_has_target_gene_1, bm_drug_has_target_gene_1,
         Ws_drug_has_target_gene_1, bs_drug_has_target_gene_1,
         Wm_gene_get_target_drug_1, bm_gene_get_target_drug_1,
         Ws_gene_get_target_drug_1, bs_gene_get_target_drug_1,
         Wm_drug_rel0_drug_1, bm_drug_rel0_drug_1,
         Ws_drug_rel0_drug_1, bs_drug_rel0_drug_1,
         Wm_drug_rel1_drug_1, bm_drug_rel1_drug_1,
         Ws_drug_rel1_drug_1, bs_drug_rel1_drug_1),
    ]
    sd_gg = _pad_edges(ei_gene_interact_gene, 200704)
    sd_dg = _pad_edges(ei_drug_has_target_gene, 100352)
    sd_gd = _pad_edges(ei_gene_get_target_drug, 100352)
    sd_d0 = _pad_edges(ei_drug_rel0_drug, 100352)
    sd_d1 = _pad_edges(ei_drug_rel1_drug, 100352)

    xg = _pad_rows(x_gene)
    xd = _pad_rows(x_drug)
    for l in range(2):
        (Wm_gg, bm_gg, Ws_gg, bs_gg,
         Wm_dg, bm_dg, Ws_dg, bs_dg,
         Wm_gd, bm_gd, Ws_gd, bs_gd,
         Wm_d0, bm_d0, Ws_d0, bs_d0,
         Wm_d1, bm_d1, Ws_d1, bs_d1) = Wl[l]
        Wg = jnp.concatenate([Wm_gg, Wm_gd, Ws_gg, Ws_dg], axis=1)
        bg = jnp.tile(jnp.concatenate([bm_gg, bm_gd, bs_gg, bs_dg])[None, :],
                      (8, 1))
        Wd = jnp.concatenate([Wm_dg, Wm_d0, Wm_d1, Ws_gd, Ws_d0, Ws_d1],
                             axis=1)
        bd = jnp.tile(
            jnp.concatenate([bm_dg, bm_d0, bm_d1, bs_gd, bs_d0, bs_d1])[None, :],
            (8, 1))
        Ymgg, Ymgd, Sgg, Sdg = _mm_call(4)(xg, Wg, bg)
        Ymdg, Ymd0, Ymd1, Sgd, Sd0, Sd1 = _mm_call(6)(xd, Wd, bd)
        agg_gg = _agg_call(200704)(sd_gg, Ymgg)
        agg_dg = _agg_call(100352)(sd_dg, Ymdg)
        agg_gd = _agg_call(100352)(sd_gd, Ymgd)
        agg_d0 = _agg_call(100352)(sd_d0, Ymd0)
        agg_d1 = _agg_call(100352)(sd_d1, Ymd1)
        relu = l == 0
        xg = _comb_call(2, relu)(agg_gg, Sgg, agg_dg, Sdg)
        xd = _comb_call(3, relu)(agg_gd, Sgd, agg_d0, Sd0, agg_d1, Sd1)

    idx_cat = jnp.concatenate([
        _pad_idx(ei_drug_rel0_drug[0]), _pad_idx(ei_drug_rel0_drug[1]),
        _pad_idx(ei_drug_rel1_drug[0]),
        _pad_idx(ei_drug_rel1_drug[1], extra=EB)])
    Zr = _gather_call(4 * GPAD)(idx_cat, xd)
    R1 = (D_dedicom_rel1[:, None] * R_dedicom) * D_dedicom_rel1[None, :]
    s0 = _dec_call(0, 49)(Zr, Zr, R_bilinear_rel0)
    s1 = _dec_call(98, 147)(Zr, Zr, R1)
    return jnp.concatenate(
        [s0.reshape(-1)[:100000], s1.reshape(-1)[:100000]])
